# baseline (speedup 1.0000x reference)
import functools

import jax
import jax.numpy as jnp
import numpy as np
from jax import lax
from jax.experimental import pallas as pl
from jax.experimental.pallas import tpu as pltpu
from jax.experimental.pallas import tpu_sc as plsc

_PRIOR = 0.1
_B, _L, _D, _H = 4, 2048, 512, 1024
_ROWT = 1024
_DBLK = 32
_NJ = _D // _DBLK
_CH = 256


def _mlp_kernel(x_ref, w1_ref, b1_ref, w2_ref, b2_ref, out_ref):
    h1 = jnp.maximum(
        jnp.dot(x_ref[...], w1_ref[...], preferred_element_type=jnp.float32)
        + b1_ref[...], 0.0)
    s = jnp.dot(h1[:, 0:256], w2_ref[0:256, :],
                preferred_element_type=jnp.float32)
    for i in range(1, _H // 256):
        s = s + jnp.dot(h1[:, i * 256:(i + 1) * 256],
                        w2_ref[i * 256:(i + 1) * 256, :],
                        preferred_element_type=jnp.float32)
    out_ref[...] = s + b2_ref[...]


def _boundary_kernel(logits_ref, mask_ref, seg_ref, short_ref, nseg_ref,
                     counts_ref, loss_ref, nb_ref, tp_ref):
    B, L = logits_ref.shape
    logits = logits_ref[...]
    mask = mask_ref[...]
    probs = jax.nn.sigmoid(logits)
    hard = jnp.where(probs > 0.5, 1.0, 0.0)
    hb = (hard + probs) - probs
    hb = hb * mask
    n = jnp.sum(mask, axis=1, keepdims=True)
    col = jax.lax.broadcasted_iota(jnp.int32, (B, L), 1).astype(jnp.float32)
    last_real = jnp.where((col == n - 1.0) & (n < float(L)), 1.0, 0.0)
    hb = jnp.maximum(hb, last_real)
    cum = hb
    for k in range(11):
        s = 1 << k
        rolled = jnp.roll(cum, s, axis=1)
        cum = cum + jnp.where(col >= float(s), rolled, 0.0)
    hh1 = cum - hb
    seg_r = jnp.round(hh1)
    drop = hh1 != seg_r
    seg_i = jnp.where(drop, jnp.int32(L), seg_r.astype(jnp.int32))
    seg_ref[...] = seg_i
    nseg = 1 + jnp.max(jnp.where(drop, -1, seg_i), axis=1, keepdims=True)
    nseg_ref[...] = jnp.broadcast_to(nseg, (B, 16))
    counts = jnp.sum(jnp.round(hb), axis=1, keepdims=True)
    counts_ref[...] = counts
    n_kept = jnp.sum(jnp.where(hb == 1.0, 1.0, 0.0), axis=1, keepdims=True)
    short_ref[...] = jnp.where(col < n_kept, 1.0, 0.0)
    nb_ref[...] = jnp.sum(counts).reshape(1, 1)
    tp_ref[...] = jnp.sum(n).reshape(1, 1)
    kval = col + 1.0
    logk = jnp.log(kval)

    def logfact(m):
        return jnp.sum(jnp.where((kval >= 2.0) & (kval <= m), logk, 0.0),
                       axis=1, keepdims=True)

    logprob = (logfact(n) - logfact(counts) - logfact(n - counts)
               + counts * np.log(_PRIOR) + (n - counts) * np.log1p(-_PRIOR))
    loss_ref[...] = (10.0 * jnp.mean(-(logprob / n))).reshape(1, 1)


def _sc_pool_kernel(hid_hbm, seg_hbm, nseg_hbm, pe_hbm, out_hbm,
                    acc, in_buf, seg_buf, pe_buf, tail_buf):
    NC = 2
    wid = lax.axis_index("s") * NC + lax.axis_index("c")
    lane = lax.iota(jnp.int32, 16)
    one8 = jnp.where(lane == 8, 1.0, 0.0)

    for u_off in range(2):
        u = wid * 2 + u_off
        b = u // _NJ
        j0 = (u % _NJ) * _DBLK

        pltpu.sync_copy(nseg_hbm.at[b], tail_buf)
        nseg = tail_buf[pl.ds(0, 16)][0]

        def zero_row(r, _):
            z = jnp.zeros((16,), jnp.float32)
            acc[r, pl.ds(0, 16)] = z
            acc[r, pl.ds(16, 16)] = z
            acc[r, pl.ds(24, 16)] = z
            return _
        lax.fori_loop(0, nseg, zero_row, 0)

        for c in range(_L // _CH):
            c0 = c * _CH
            pltpu.sync_copy(hid_hbm.at[b, pl.ds(c0, _CH), pl.ds(j0, _DBLK)],
                            in_buf)
            pltpu.sync_copy(seg_hbm.at[b, pl.ds(c0, _CH)], seg_buf)

            def group(g, _):
                sv = seg_buf[pl.ds(g * 16, 16)]
                for i in range(16):
                    p = g * 16 + i
                    srow = sv[i]
                    acc[srow, pl.ds(0, 16)] = (
                        acc[srow, pl.ds(0, 16)] + in_buf[p, pl.ds(0, 16)])
                    acc[srow, pl.ds(16, 16)] = (
                        acc[srow, pl.ds(16, 16)] + in_buf[p, pl.ds(16, 16)])
                    acc[srow, pl.ds(24, 16)] = acc[srow, pl.ds(24, 16)] + one8
                return _
            lax.fori_loop(0, _CH // 16, group, 0)

        for c in range(_L // _CH):
            s0 = c * _CH
            pltpu.sync_copy(pe_hbm.at[pl.ds(s0, _CH), pl.ds(j0, _DBLK)],
                            pe_buf)
            nrows = jnp.clip(nseg - s0, 0, _CH)

            def mean_row(r, _):
                srow = s0 + r
                cntv = jnp.full((16,), acc[srow, pl.ds(24, 16)][8],
                                jnp.float32)
                inv = 1.0 / (cntv + 1e-9)
                pe_buf[r, pl.ds(0, 16)] = (
                    acc[srow, pl.ds(0, 16)] * inv + pe_buf[r, pl.ds(0, 16)])
                pe_buf[r, pl.ds(16, 16)] = (
                    acc[srow, pl.ds(16, 16)] * inv + pe_buf[r, pl.ds(16, 16)])
                return _
            lax.fori_loop(0, nrows, mean_row, 0)
            pltpu.sync_copy(pe_buf,
                            out_hbm.at[b, pl.ds(s0, _CH), pl.ds(j0, _DBLK)])


def _pos_emb(S, D):
    pos = jnp.arange(S, dtype=jnp.float32)[:, None]
    i = jnp.arange(0, D, 2, dtype=jnp.float32)[None, :]
    div = jnp.exp(-(jnp.log(10000.0)) * i / D)
    pe = jnp.zeros((S, D), dtype=jnp.float32)
    pe = pe.at[:, 0::2].set(jnp.sin(pos * div))
    pe = pe.at[:, 1::2].set(jnp.cos(pos * div))
    return pe


@jax.jit
def kernel(hidden, attention_mask, W1, b1, W2, b2):
    B, L, D = hidden.shape
    H = W1.shape[1]
    f32 = jnp.float32

    logits = pl.pallas_call(
        _mlp_kernel,
        grid=(B * L // _ROWT,),
        in_specs=[
            pl.BlockSpec((_ROWT, D), lambda i: (i, 0)),
            pl.BlockSpec((D, H), lambda i: (0, 0)),
            pl.BlockSpec((1, H), lambda i: (0, 0)),
            pl.BlockSpec((H, 1), lambda i: (0, 0)),
            pl.BlockSpec((1, 1), lambda i: (0, 0)),
        ],
        out_specs=pl.BlockSpec((_ROWT, 1), lambda i: (i, 0)),
        out_shape=jax.ShapeDtypeStruct((B * L, 1), f32),
    )(hidden.reshape(B * L, D), W1, b1.reshape(1, H), W2, b2.reshape(1, 1))
    logits = logits.reshape(B, L)

    seg, short_mask, nseg_arr, counts, loss, nb, tp = pl.pallas_call(
        _boundary_kernel,
        out_shape=(
            jax.ShapeDtypeStruct((B, L), jnp.int32),
            jax.ShapeDtypeStruct((B, L), f32),
            jax.ShapeDtypeStruct((B, 16), jnp.int32),
            jax.ShapeDtypeStruct((B, 1), f32),
            jax.ShapeDtypeStruct((1, 1), f32),
            jax.ShapeDtypeStruct((1, 1), f32),
            jax.ShapeDtypeStruct((1, 1), f32),
        ),
    )(logits, attention_mask)

    pe = _pos_emb(L, D)
    mesh = plsc.VectorSubcoreMesh(core_axis_name="c", subcore_axis_name="s")
    pooled = pl.kernel(
        _sc_pool_kernel,
        mesh=mesh,
        compiler_params=pltpu.CompilerParams(use_tc_tiling_on_sc=False),
        out_type=jax.ShapeDtypeStruct((B, L, D), f32),
        scratch_types=[
            pltpu.VMEM((L + 1, 40), f32),
            pltpu.VMEM((_CH, _DBLK), f32),
            pltpu.VMEM((_CH,), jnp.int32),
            pltpu.VMEM((_CH, _DBLK), f32),
            pltpu.VMEM((16,), jnp.int32),
        ],
    )(hidden, seg, nseg_arr, pe)

    return (pooled, loss[0, 0], nb[0, 0], tp[0, 0], short_mask)
